# baseline (speedup 1.0000x reference)
import functools

import jax
import jax.numpy as jnp
from jax import lax
from jax.experimental import pallas as pl
from jax.experimental.pallas import tpu as pltpu
from jax.experimental.pallas import tpu_sc as plsc

H = 128
L = 16
NCH = H // L
NSUB = 16
NCORE = 2
CHUNKS_PER_CORE = NCH // NCORE

_N = {'operator': 100000, 'table': 50000, 'column': 100000, 'predicate': 80000}
_IN_DIMS = {'operator': 4, 'table': 2, 'column': 8, 'predicate': 1}
_ETYPES = [('table', 'operator', 'scannedby'),
           ('predicate', 'operator', 'filters'),
           ('column', 'operator', 'outputby'),
           ('column', 'predicate', 'connects'),
           ('operator', 'operator', 'calledby'),
           ('table', 'table', 'selfloop_table'),
           ('column', 'column', 'selfloop_column')]
_ECNT = {'scannedby': 100000, 'filters': 100000, 'outputby': 100000,
         'connects': 100000, 'calledby': 100000, 'selfloop_table': 50000,
         'selfloop_column': 100000}
NUM_GRAPHS = 64
_NTYPES = ['operator', 'table', 'column', 'predicate']


def _ceil_to(x, m):
    return (x + m - 1) // m * m


_NPAD = {t: _ceil_to(_N[t] + 1, 2048) for t in _NTYPES}
_EPAD = {n: _ceil_to(_ECNT[n], 2048) for n in _ECNT}
_ACC_ROWS = max(_NPAD.values())
_EROWS_MAX = _ceil_to(max(_EPAD.values()) // (NSUB * 128), 8)


_KI = 4


def _proj_multi(x, wts, bs, relu_in=False, bn=2048):
    np_rows = x.shape[0]
    grid = (np_rows // bn,)
    k = len(wts)

    def body(*refs):
        x_ref = refs[0]
        w_refs = refs[1:1 + k]
        b_refs = refs[1 + k:1 + 2 * k]
        o_refs = refs[1 + 2 * k:]
        xb = x_ref[...]
        if relu_in:
            xb = jnp.maximum(xb, 0.0)
        for w_ref, b_ref, o_ref in zip(w_refs, b_refs, o_refs):
            y = jnp.dot(xb, w_ref[...], preferred_element_type=jnp.float32)
            o_ref[...] = y + b_ref[...]

    return pl.pallas_call(
        body, grid=grid,
        in_specs=([pl.BlockSpec((bn, x.shape[1]), lambda i: (i, 0))]
                  + [pl.BlockSpec(w.shape, lambda i: (0, 0)) for w in wts]
                  + [pl.BlockSpec((1, H), lambda i: (0, 0)) for _ in bs]),
        out_specs=[pl.BlockSpec((bn, H), lambda i: (i, 0)) for _ in wts],
        out_shape=[jax.ShapeDtypeStruct((np_rows, H), jnp.float32)
                   for _ in wts],
    )(x, *wts, *bs)


def _pool_kernel(x, ids3, wt, b, bn=2048):
    np_rows = x.shape[0]
    ngrid = np_rows // bn

    def body(x_ref, ids_ref, w_ref, b_ref, o_ref, acc, cnt):
        i = pl.program_id(0)

        @pl.when(i == 0)
        def _():
            acc[...] = jnp.zeros_like(acc)
            cnt[...] = jnp.zeros_like(cnt)

        xb = jnp.maximum(x_ref[...], 0.0)
        ids = ids_ref[0]
        gids = lax.broadcasted_iota(jnp.int32, (NUM_GRAPHS, bn), 0)
        oh = (gids == ids).astype(jnp.float32)
        acc[...] += jnp.dot(oh, xb, preferred_element_type=jnp.float32)
        cnt[...] += jnp.sum(oh, axis=1, keepdims=True)

        @pl.when(i == ngrid - 1)
        def _():
            pooled = acc[...] / jnp.maximum(cnt[...], 1.0)
            o_ref[...] = jnp.dot(pooled, w_ref[...],
                                 preferred_element_type=jnp.float32) + b_ref[...]

    return pl.pallas_call(
        body, grid=(ngrid,),
        in_specs=[
            pl.BlockSpec((bn, H), lambda i: (i, 0)),
            pl.BlockSpec((1, 1, bn), lambda i: (i, 0, 0)),
            pl.BlockSpec(wt.shape, lambda i: (0, 0)),
            pl.BlockSpec((1, 1), lambda i: (0, 0)),
        ],
        out_specs=pl.BlockSpec((NUM_GRAPHS, 1), lambda i: (0, 0)),
        out_shape=jax.ShapeDtypeStruct((NUM_GRAPHS, 1), jnp.float32),
        scratch_shapes=[
            pltpu.VMEM((NUM_GRAPHS, H), jnp.float32),
            pltpu.VMEM((NUM_GRAPHS, 1), jnp.float32),
        ],
    )(x, ids3, wt, b)


def _sc_layer(dst_specs, rel_erows, inits, zs, srcs, dsts):
    ntypes = len(dst_specs)
    nrels = len(zs)
    mesh = plsc.VectorSubcoreMesh(core_axis_name="c", subcore_axis_name="s",
                                  num_cores=NCORE, num_subcores=NSUB)

    @functools.partial(
        pl.kernel,
        out_type=[jax.ShapeDtypeStruct((sp[0], H), jnp.float32)
                  for sp in dst_specs],
        mesh=mesh,
        scratch_types=[
            pltpu.VMEM_SHARED((_ACC_ROWS, L), jnp.float32),
            pltpu.VMEM((_EROWS_MAX, 128), jnp.int32),
            pltpu.VMEM((_EROWS_MAX, 128), jnp.int32),
            pltpu.VMEM((2 * _KI * 128, L), jnp.float32),
            pltpu.SemaphoreType.DMA,
            pltpu.SemaphoreType.DMA,
        ],
        compiler_params=pltpu.CompilerParams(use_tc_tiling_on_sc=False),
    )
    def kfn(*refs):
        init_refs = refs[:ntypes]
        z_refs = refs[ntypes:ntypes + nrels]
        s_refs = refs[ntypes + nrels:ntypes + 2 * nrels]
        d_refs = refs[ntypes + 2 * nrels:ntypes + 3 * nrels]
        out_refs = refs[ntypes + 3 * nrels:ntypes + 3 * nrels + ntypes]
        acc, sidx, didx, rows, gsem, ssem = refs[ntypes + 3 * nrels + ntypes:]

        cid = lax.axis_index("c")
        sid = lax.axis_index("s")

        for ti, (nt_pad, rel_ids) in enumerate(dst_specs):
            rpw = nt_pad // NSUB
            for cc in range(CHUNKS_PER_CORE):
                ch = cid * CHUNKS_PER_CORE + cc
                pltpu.sync_copy(
                    init_refs[ti].at[pl.ds(sid * rpw, rpw), pl.ds(ch * L, L)],
                    acc.at[pl.ds(sid * rpw, rpw)])
                plsc.subcore_barrier()
                for r in rel_ids:
                    erows = _ceil_to(rel_erows[r], _KI)
                    stride = s_refs[r].shape[1] // NSUB
                    assert erows <= stride
                    ki = _KI
                    nouter = erows // ki
                    pltpu.sync_copy(
                        s_refs[r].at[ch].at[pl.ds(sid * stride, stride)],
                        sidx.at[pl.ds(0, stride)])
                    pltpu.sync_copy(d_refs[r].at[pl.ds(sid * stride, stride)],
                                    didx.at[pl.ds(0, stride)])

                    kb = ki * 128

                    def outer(o, _, r=r, ki=ki, kb=kb):
                        off = (o % 2) * kb

                        @pl.when(o >= 2)
                        def _():
                            pltpu.make_async_copy(
                                z_refs[r].at[pl.ds(0, kb)],
                                rows.at[pl.ds(off, kb)], ssem).wait()

                        descs = []
                        for j in range(ki):
                            descs.append(pltpu.async_copy(
                                z_refs[r].at[sidx.at[o * ki + j]],
                                rows.at[pl.ds(off + j * 128, 128)], gsem))
                        for dsc in descs:
                            dsc.wait()
                        for j in range(ki):
                            pltpu.async_copy(
                                rows.at[pl.ds(off + j * 128, 128)],
                                acc.at[didx.at[o * ki + j]], ssem, add=True)
                        return 0

                    lax.fori_loop(0, nouter, outer, 0)
                    for b in range(min(nouter, 2)):
                        pltpu.make_async_copy(
                            z_refs[r].at[pl.ds(0, kb)],
                            rows.at[pl.ds(b * kb, kb)], ssem).wait()
                plsc.subcore_barrier()
                pltpu.sync_copy(
                    acc.at[pl.ds(sid * rpw, rpw)],
                    out_refs[ti].at[pl.ds(sid * rpw, rpw), pl.ds(ch * L, L)])
                plsc.subcore_barrier()

    return kfn(*(list(inits) + list(zs) + list(srcs) + list(dsts)))


def kernel(x_operator, x_table, x_column, x_predicate, params,
           edge_index_scannedby, edge_index_filters, edge_index_outputby,
           edge_index_connects, edge_index_calledby,
           edge_index_selfloop_table, edge_index_selfloop_column,
           batch_operator):
    xs = {'operator': x_operator, 'table': x_table,
          'column': x_column, 'predicate': x_predicate}
    edges = {'scannedby': edge_index_scannedby, 'filters': edge_index_filters,
             'outputby': edge_index_outputby, 'connects': edge_index_connects,
             'calledby': edge_index_calledby,
             'selfloop_table': edge_index_selfloop_table,
             'selfloop_column': edge_index_selfloop_column}
    p = params

    xp = {t: jnp.pad(xs[t], ((0, _NPAD[t] - _N[t]), (0, 0))) for t in _NTYPES}
    srcp, dstp, erows_d = {}, {}, {}
    for (st, dt, name) in _ETYPES:
        e = _ECNT[name]
        ep = _EPAD[name]
        erows = ep // (NSUB * 128)
        stride = _ceil_to(erows, 8)
        erows_d[name] = erows

        def _lay(v, fill):
            v = jnp.pad(v, (0, ep - e), constant_values=fill)
            v = v.reshape(NSUB, erows, 128)
            v = jnp.pad(v, ((0, 0), (0, stride - erows), (0, 0)),
                        constant_values=fill)
            return v.reshape(NSUB * stride, 128)

        s0 = _lay(edges[name][0], 0)
        srcp[name] = (s0[None] * NCH
                      + jnp.arange(NCH, dtype=jnp.int32)[:, None, None])
        dstp[name] = _lay(edges[name][1], _N[dt])

    wlin = {t: p['lin_%s_W' % t] for t in _NTYPES}
    blin = {t: p['lin_%s_b' % t] for t in _NTYPES}

    z1, init1 = {}, {}
    for t in _NTYPES:
        names = [name for (st, dt, name) in _ETYPES if st == t]
        wts, bs = [], []
        for name in names:
            wrel = p['c1_%s_Wrel' % name]
            wts.append((wrel @ wlin[t]).T)
            bs.append((wrel @ blin[t]).reshape(1, H))
        rels_t = [name for (st, dt, name) in _ETYPES if dt == t]
        wroot = sum(p['c1_%s_Wroot' % name] for name in rels_t)
        brel = sum(p['c1_%s_brel' % name] for name in rels_t)
        wts.append((wroot @ wlin[t]).T)
        bs.append((wroot @ blin[t] + brel).reshape(1, H))
        outs = _proj_multi(xp[t], wts, bs)
        for name, o in zip(names, outs[:-1]):
            z1[name] = o.reshape(_NPAD[t] * NCH, L)
        init1[t] = outs[-1]

    rel_order = [name for (_, _, name) in _ETYPES]
    dst_specs1 = []
    for t in _NTYPES:
        rel_ids = [i for i, (st, dt, name) in enumerate(_ETYPES) if dt == t]
        dst_specs1.append((_NPAD[t], rel_ids))
    outs1 = _sc_layer(dst_specs1,
                      [erows_d[name] for name in rel_order],
                      [init1[t] for t in _NTYPES],
                      [z1[name] for name in rel_order],
                      [srcp[name] for name in rel_order],
                      [dstp[name] for name in rel_order])
    out1 = dict(zip(_NTYPES, outs1))

    l2_rels = [(st, dt, name) for (st, dt, name) in _ETYPES if dt == 'operator']
    wroot2 = sum(p['c2_%s_Wroot' % name] for (_, _, name) in l2_rels)
    brel2 = sum(p['c2_%s_brel' % name] for (_, _, name) in l2_rels)
    z2d, init2 = {}, None
    for t in _NTYPES:
        names = [name for (st, dt, name) in l2_rels if st == t]
        if not names:
            continue
        wts = [p['c2_%s_Wrel' % name].T for name in names]
        bs = [jnp.zeros((1, H), jnp.float32) for _ in names]
        if t == 'operator':
            wts.append(wroot2.T)
            bs.append(brel2.reshape(1, H))
        outs = _proj_multi(out1[t], wts, bs, relu_in=True)
        for name, o in zip(names, outs):
            z2d[name] = o.reshape(_NPAD[t] * NCH, L)
        if t == 'operator':
            init2 = outs[-1]
    z2 = [z2d[name] for (_, _, name) in l2_rels]
    s2 = [srcp[name] for (_, _, name) in l2_rels]
    d2 = [dstp[name] for (_, _, name) in l2_rels]
    dst_specs2 = [(_NPAD['operator'], list(range(len(l2_rels))))]
    erows2 = [erows_d[name] for (_, _, name) in l2_rels]
    (out2_op,) = _sc_layer(dst_specs2, erows2, [init2], z2, s2, d2)

    ids = jnp.pad(batch_operator, (0, _NPAD['operator'] - _N['operator']),
                  constant_values=NUM_GRAPHS + 1)
    ids3 = ids.reshape(_NPAD['operator'] // 2048, 1, 2048)
    res = _pool_kernel(out2_op, ids3,
                       p['lin_out_W'].T, p['lin_out_b'].reshape(1, 1))
    return res.reshape(NUM_GRAPHS)


# speedup vs baseline: 1.9032x; 1.9032x over previous
import functools

import jax
import jax.numpy as jnp
from jax import lax
from jax.experimental import pallas as pl
from jax.experimental.pallas import tpu as pltpu
from jax.experimental.pallas import tpu_sc as plsc

H = 128
L = 16
NCH = H // L
NSUB = 16
NCORE = 2
CHUNKS_PER_CORE = NCH // NCORE

_N = {'operator': 100000, 'table': 50000, 'column': 100000, 'predicate': 80000}
_IN_DIMS = {'operator': 4, 'table': 2, 'column': 8, 'predicate': 1}
_ETYPES = [('table', 'operator', 'scannedby'),
           ('predicate', 'operator', 'filters'),
           ('column', 'operator', 'outputby'),
           ('column', 'predicate', 'connects'),
           ('operator', 'operator', 'calledby'),
           ('table', 'table', 'selfloop_table'),
           ('column', 'column', 'selfloop_column')]
_ECNT = {'scannedby': 100000, 'filters': 100000, 'outputby': 100000,
         'connects': 100000, 'calledby': 100000, 'selfloop_table': 50000,
         'selfloop_column': 100000}
NUM_GRAPHS = 64
_NTYPES = ['operator', 'table', 'column', 'predicate']


def _ceil_to(x, m):
    return (x + m - 1) // m * m


_NPAD = {t: _ceil_to(_N[t] + 1, 2048) for t in _NTYPES}
_EPAD = {n: _ceil_to(_ECNT[n], 2048) for n in _ECNT}
_ACC_ROWS = max(_NPAD.values())
_EROWS_MAX = _ceil_to(max(_EPAD.values()) // (NSUB * 128), 8)


def _inner_k(nrows):
    for k in (8, 7, 5, 6, 4, 3, 2):
        if nrows % k == 0:
            return k
    return 1


def _proj_multi(x, wts, bs, relu_in=False, bn=2048):
    np_rows = x.shape[0]
    grid = (np_rows // bn,)
    k = len(wts)

    def body(*refs):
        x_ref = refs[0]
        w_refs = refs[1:1 + k]
        b_refs = refs[1 + k:1 + 2 * k]
        o_refs = refs[1 + 2 * k:]
        xb = x_ref[...]
        if relu_in:
            xb = jnp.maximum(xb, 0.0)
        for w_ref, b_ref, o_ref in zip(w_refs, b_refs, o_refs):
            y = jnp.dot(xb, w_ref[...], preferred_element_type=jnp.float32)
            o_ref[...] = y + b_ref[...]

    return pl.pallas_call(
        body, grid=grid,
        in_specs=([pl.BlockSpec((bn, x.shape[1]), lambda i: (i, 0))]
                  + [pl.BlockSpec(w.shape, lambda i: (0, 0)) for w in wts]
                  + [pl.BlockSpec((1, H), lambda i: (0, 0)) for _ in bs]),
        out_specs=[pl.BlockSpec((bn, H), lambda i: (i, 0)) for _ in wts],
        out_shape=[jax.ShapeDtypeStruct((np_rows, H), jnp.float32)
                   for _ in wts],
    )(x, *wts, *bs)


def _pool_kernel(x, ids3, wt, b, bn=2048):
    np_rows = x.shape[0]
    ngrid = np_rows // bn

    def body(x_ref, ids_ref, w_ref, b_ref, o_ref, acc, cnt):
        i = pl.program_id(0)

        @pl.when(i == 0)
        def _():
            acc[...] = jnp.zeros_like(acc)
            cnt[...] = jnp.zeros_like(cnt)

        xb = jnp.maximum(x_ref[...], 0.0)
        ids = ids_ref[0]
        gids = lax.broadcasted_iota(jnp.int32, (NUM_GRAPHS, bn), 0)
        oh = (gids == ids).astype(jnp.float32)
        acc[...] += jnp.dot(oh, xb, preferred_element_type=jnp.float32)
        cnt[...] += jnp.sum(oh, axis=1, keepdims=True)

        @pl.when(i == ngrid - 1)
        def _():
            pooled = acc[...] / jnp.maximum(cnt[...], 1.0)
            o_ref[...] = jnp.dot(pooled, w_ref[...],
                                 preferred_element_type=jnp.float32) + b_ref[...]

    return pl.pallas_call(
        body, grid=(ngrid,),
        in_specs=[
            pl.BlockSpec((bn, H), lambda i: (i, 0)),
            pl.BlockSpec((1, 1, bn), lambda i: (i, 0, 0)),
            pl.BlockSpec(wt.shape, lambda i: (0, 0)),
            pl.BlockSpec((1, 1), lambda i: (0, 0)),
        ],
        out_specs=pl.BlockSpec((NUM_GRAPHS, 1), lambda i: (0, 0)),
        out_shape=jax.ShapeDtypeStruct((NUM_GRAPHS, 1), jnp.float32),
        scratch_shapes=[
            pltpu.VMEM((NUM_GRAPHS, H), jnp.float32),
            pltpu.VMEM((NUM_GRAPHS, 1), jnp.float32),
        ],
    )(x, ids3, wt, b)


def _sc_layer(dst_specs, rel_erows, inits, zs, srcs, dsts):
    ntypes = len(dst_specs)
    nrels = len(zs)
    mesh = plsc.VectorSubcoreMesh(core_axis_name="c", subcore_axis_name="s",
                                  num_cores=NCORE, num_subcores=NSUB)

    @functools.partial(
        pl.kernel,
        out_type=[jax.ShapeDtypeStruct((sp[0], H), jnp.float32)
                  for sp in dst_specs],
        mesh=mesh,
        scratch_types=[
            pltpu.VMEM_SHARED((_ACC_ROWS, L), jnp.float32),
            pltpu.VMEM((_EROWS_MAX, 128), jnp.int32),
            pltpu.VMEM((_EROWS_MAX, 128), jnp.int32),
            pltpu.VMEM((8 * 128, L), jnp.float32),
            pltpu.SemaphoreType.DMA,
            pltpu.SemaphoreType.DMA,
        ],
        compiler_params=pltpu.CompilerParams(use_tc_tiling_on_sc=False),
    )
    def kfn(*refs):
        init_refs = refs[:ntypes]
        z_refs = refs[ntypes:ntypes + nrels]
        s_refs = refs[ntypes + nrels:ntypes + 2 * nrels]
        d_refs = refs[ntypes + 2 * nrels:ntypes + 3 * nrels]
        out_refs = refs[ntypes + 3 * nrels:ntypes + 3 * nrels + ntypes]
        acc, sidx, didx, rows, gsem, ssem = refs[ntypes + 3 * nrels + ntypes:]

        cid = lax.axis_index("c")
        sid = lax.axis_index("s")

        for ti, (nt_pad, rel_ids) in enumerate(dst_specs):
            rpw = nt_pad // NSUB
            for cc in range(CHUNKS_PER_CORE):
                ch = cid * CHUNKS_PER_CORE + cc
                pltpu.sync_copy(
                    init_refs[ti].at[pl.ds(sid * rpw, rpw), pl.ds(ch * L, L)],
                    acc.at[pl.ds(sid * rpw, rpw)])
                plsc.subcore_barrier()
                for r in rel_ids:
                    erows = rel_erows[r]
                    stride = s_refs[r].shape[1] // NSUB
                    ki = _inner_k(erows)
                    nouter = erows // ki
                    pltpu.sync_copy(
                        s_refs[r].at[ch].at[pl.ds(sid * stride, stride)],
                        sidx.at[pl.ds(0, stride)])
                    pltpu.sync_copy(d_refs[r].at[pl.ds(sid * stride, stride)],
                                    didx.at[pl.ds(0, stride)])

                    def outer(o, _, r=r, ki=ki):
                        gds, sds = [], []
                        for j in range(ki):
                            gds.append(pltpu.async_copy(
                                z_refs[r].at[sidx.at[o * ki + j]],
                                rows.at[pl.ds(j * 128, 128)], gsem))
                        for dsc in gds:
                            dsc.wait()
                        for j in range(ki):
                            sds.append(pltpu.async_copy(
                                rows.at[pl.ds(j * 128, 128)],
                                acc.at[didx.at[o * ki + j]], ssem, add=True))
                        for dsc in sds:
                            dsc.wait()
                        return 0

                    lax.fori_loop(0, nouter, outer, 0)
                plsc.subcore_barrier()
                pltpu.sync_copy(
                    acc.at[pl.ds(sid * rpw, rpw)],
                    out_refs[ti].at[pl.ds(sid * rpw, rpw), pl.ds(ch * L, L)])
                plsc.subcore_barrier()

    return kfn(*(list(inits) + list(zs) + list(srcs) + list(dsts)))


def kernel(x_operator, x_table, x_column, x_predicate, params,
           edge_index_scannedby, edge_index_filters, edge_index_outputby,
           edge_index_connects, edge_index_calledby,
           edge_index_selfloop_table, edge_index_selfloop_column,
           batch_operator):
    xs = {'operator': x_operator, 'table': x_table,
          'column': x_column, 'predicate': x_predicate}
    edges = {'scannedby': edge_index_scannedby, 'filters': edge_index_filters,
             'outputby': edge_index_outputby, 'connects': edge_index_connects,
             'calledby': edge_index_calledby,
             'selfloop_table': edge_index_selfloop_table,
             'selfloop_column': edge_index_selfloop_column}
    p = params

    xp = {t: jnp.pad(xs[t], ((0, _NPAD[t] - _N[t]), (0, 0))) for t in _NTYPES}
    srcp, dstp, erows_d = {}, {}, {}
    for (st, dt, name) in _ETYPES:
        e = _ECNT[name]
        ep = _EPAD[name]
        erows = ep // (NSUB * 128)
        stride = _ceil_to(erows, 8)
        erows_d[name] = erows

        def _lay(v, fill):
            v = jnp.pad(v, (0, ep - e), constant_values=fill)
            v = v.reshape(NSUB, erows, 128)
            v = jnp.pad(v, ((0, 0), (0, stride - erows), (0, 0)),
                        constant_values=fill)
            return v.reshape(NSUB * stride, 128)

        s0 = _lay(edges[name][0], 0)
        srcp[name] = (s0[None] * NCH
                      + jnp.arange(NCH, dtype=jnp.int32)[:, None, None])
        dstp[name] = _lay(edges[name][1], _N[dt])

    wlin = {t: p['lin_%s_W' % t] for t in _NTYPES}
    blin = {t: p['lin_%s_b' % t] for t in _NTYPES}

    z1, init1 = {}, {}
    for t in _NTYPES:
        names = [name for (st, dt, name) in _ETYPES if st == t]
        wts, bs = [], []
        for name in names:
            wrel = p['c1_%s_Wrel' % name]
            wts.append((wrel @ wlin[t]).T)
            bs.append((wrel @ blin[t]).reshape(1, H))
        rels_t = [name for (st, dt, name) in _ETYPES if dt == t]
        wroot = sum(p['c1_%s_Wroot' % name] for name in rels_t)
        brel = sum(p['c1_%s_brel' % name] for name in rels_t)
        wts.append((wroot @ wlin[t]).T)
        bs.append((wroot @ blin[t] + brel).reshape(1, H))
        outs = _proj_multi(xp[t], wts, bs)
        for name, o in zip(names, outs[:-1]):
            z1[name] = o.reshape(_NPAD[t] * NCH, L)
        init1[t] = outs[-1]

    rel_order = [name for (_, _, name) in _ETYPES]
    dst_specs1 = []
    for t in _NTYPES:
        rel_ids = [i for i, (st, dt, name) in enumerate(_ETYPES) if dt == t]
        dst_specs1.append((_NPAD[t], rel_ids))
    outs1 = _sc_layer(dst_specs1,
                      [erows_d[name] for name in rel_order],
                      [init1[t] for t in _NTYPES],
                      [z1[name] for name in rel_order],
                      [srcp[name] for name in rel_order],
                      [dstp[name] for name in rel_order])
    out1 = dict(zip(_NTYPES, outs1))

    l2_rels = [(st, dt, name) for (st, dt, name) in _ETYPES if dt == 'operator']
    wroot2 = sum(p['c2_%s_Wroot' % name] for (_, _, name) in l2_rels)
    brel2 = sum(p['c2_%s_brel' % name] for (_, _, name) in l2_rels)
    z2d, init2 = {}, None
    for t in _NTYPES:
        names = [name for (st, dt, name) in l2_rels if st == t]
        if not names:
            continue
        wts = [p['c2_%s_Wrel' % name].T for name in names]
        bs = [jnp.zeros((1, H), jnp.float32) for _ in names]
        if t == 'operator':
            wts.append(wroot2.T)
            bs.append(brel2.reshape(1, H))
        outs = _proj_multi(out1[t], wts, bs, relu_in=True)
        for name, o in zip(names, outs):
            z2d[name] = o.reshape(_NPAD[t] * NCH, L)
        if t == 'operator':
            init2 = outs[-1]
    z2 = [z2d[name] for (_, _, name) in l2_rels]
    s2 = [srcp[name] for (_, _, name) in l2_rels]
    d2 = [dstp[name] for (_, _, name) in l2_rels]
    dst_specs2 = [(_NPAD['operator'], list(range(len(l2_rels))))]
    erows2 = [erows_d[name] for (_, _, name) in l2_rels]
    (out2_op,) = _sc_layer(dst_specs2, erows2, [init2], z2, s2, d2)

    ids = jnp.pad(batch_operator, (0, _NPAD['operator'] - _N['operator']),
                  constant_values=NUM_GRAPHS + 1)
    ids3 = ids.reshape(_NPAD['operator'] // 2048, 1, 2048)
    res = _pool_kernel(out2_op, ids3,
                       p['lin_out_W'].T, p['lin_out_b'].reshape(1, 1))
    return res.reshape(NUM_GRAPHS)
